# Initial kernel scaffold; baseline (speedup 1.0000x reference)
import jax
import jax.numpy as jnp
from jax.experimental import pallas as pl


def kernel(x, w1_shared, w2_shared, w3_shared, w1_routed, w2_routed, w3_routed, gate_w, gate_b):
    raise NotImplementedError("write your pallas kernel here")


# speedup vs baseline: 22.6757x; 22.6757x over previous
import functools

import jax
import jax.numpy as jnp
from jax.experimental import pallas as pl
from jax.experimental.pallas import tpu as pltpu

_B, _T, _D = 8, 4, 2048
_E, _TOPK, _I, _NS = 8, 2, 512, 2
_N = _B * _T
_SLOTS = _NS + _E


def _moe_body(x_ref, gwT_ref, gb_ref, w1s_ref, w3s_ref, w2s_ref,
              w1r_ref, w3r_ref, w2r_ref, out_ref, scores_ref, g_ref):
    s = pl.program_id(0)
    xv = x_ref[...]

    @pl.when(s == 0)
    def _gate():
        logits = jnp.dot(xv, gwT_ref[...], preferred_element_type=jnp.float32)
        scores = jax.nn.sigmoid(logits) + gb_ref[...]
        scores_ref[...] = scores
        iota = jax.lax.broadcasted_iota(jnp.int32, (_N, _E), 1)
        t1v = jnp.max(scores, axis=1, keepdims=True)
        t1i = jnp.min(jnp.where(scores == t1v, iota, _E), axis=1, keepdims=True)
        masked = jnp.where(iota == t1i, -jnp.inf, scores)
        t2v = jnp.max(masked, axis=1, keepdims=True)
        t2i = jnp.min(jnp.where(masked == t2v, iota, _E), axis=1, keepdims=True)
        denom = t1v + t2v
        g_ref[...] = (jnp.where(iota == t1i, t1v / denom, 0.0)
                      + jnp.where(iota == t2i, t2v / denom, 0.0))
        out_ref[...] = jnp.zeros_like(out_ref)

    @pl.when(s < _NS)
    def _shared():
        h = jax.nn.silu(jnp.dot(xv, w1s_ref[...], preferred_element_type=jnp.float32)
                        * jnp.dot(xv, w3s_ref[...], preferred_element_type=jnp.float32))
        out_ref[...] += jnp.dot(h, w2s_ref[...], preferred_element_type=jnp.float32)

    @pl.when(s >= _NS)
    def _routed():
        iota = jax.lax.broadcasted_iota(jnp.int32, (_N, _E), 1)
        wtok = jnp.sum(jnp.where(iota == s - _NS, g_ref[...], 0.0),
                       axis=1, keepdims=True)
        h = jax.nn.silu(jnp.dot(xv, w1r_ref[0], preferred_element_type=jnp.float32)
                        * jnp.dot(xv, w3r_ref[0], preferred_element_type=jnp.float32))
        out_ref[...] += jnp.dot(wtok * h, w2r_ref[0], preferred_element_type=jnp.float32)


@functools.partial(jax.jit, static_argnames=())
def kernel(x, w1_shared, w2_shared, w3_shared, w1_routed, w2_routed, w3_routed,
           gate_w, gate_b):
    x2d = x.reshape(_N, _D)
    gwT = gate_w.T
    gb = gate_b.reshape(1, _E)

    def _c(i):
        return jnp.clip(i - _NS, 0, _E - 1)

    out2d, scores2d = pl.pallas_call(
        _moe_body,
        grid=(_SLOTS,),
        in_specs=[
            pl.BlockSpec((_N, _D), lambda i: (0, 0)),
            pl.BlockSpec((_D, _E), lambda i: (0, 0)),
            pl.BlockSpec((1, _E), lambda i: (0, 0)),
            pl.BlockSpec((_D, _I), lambda i: (0, jnp.clip(i, 0, _NS - 1))),
            pl.BlockSpec((_D, _I), lambda i: (0, jnp.clip(i, 0, _NS - 1))),
            pl.BlockSpec((_I, _D), lambda i: (jnp.clip(i, 0, _NS - 1), 0)),
            pl.BlockSpec((1, _D, _I), lambda i: (_c(i), 0, 0)),
            pl.BlockSpec((1, _D, _I), lambda i: (_c(i), 0, 0)),
            pl.BlockSpec((1, _I, _D), lambda i: (_c(i), 0, 0)),
        ],
        out_specs=[
            pl.BlockSpec((_N, _D), lambda i: (0, 0)),
            pl.BlockSpec((_N, _E), lambda i: (0, 0)),
        ],
        out_shape=[
            jax.ShapeDtypeStruct((_N, _D), jnp.float32),
            jax.ShapeDtypeStruct((_N, _E), jnp.float32),
        ],
        scratch_shapes=[pltpu.VMEM((_N, _E), jnp.float32)],
        compiler_params=pltpu.CompilerParams(
            dimension_semantics=("arbitrary",),
        ),
    )(x2d, gwT, gb, w1_shared, w3_shared, w2_shared,
      w1_routed, w3_routed, w2_routed)

    return (out2d.reshape(_B, _T, _D), scores2d.reshape(_B, _T, _E))
